# baseline (speedup 1.0000x reference)
import functools

import jax
import jax.numpy as jnp
from jax import lax
from jax.experimental import pallas as pl
from jax.experimental.pallas import tpu as pltpu
from jax.experimental.pallas import tpu_sc as plsc

_N, _C, _H, _W = 4, 96, 224, 224
_B = _N * _H * _W
_NW = 32
_PW = _B // _NW
_CH = 128
_NCHUNK = _PW // _CH
_L = 16


def _warp_body(table, gx_hbm, gy_hbm, out_hbm,
               gx_v, gy_v,
               i00, i01, i10, i11,
               w00, w01, w10, w11,
               r00, r01, r10, r11,
               out_v, sem):
    wid = lax.axis_index("s") * 2 + lax.axis_index("c")
    base = wid * _PW
    nb = (wid // 8) * (_H * _W)

    def chunk_body(ch, _):
        start = base + ch * _CH
        pltpu.sync_copy(gx_hbm.at[pl.ds(start, _CH)], gx_v)
        pltpu.sync_copy(gy_hbm.at[pl.ds(start, _CH)], gy_v)
        for i in range(_CH // _L):
            s = i * _L
            gxv = gx_v[pl.ds(s, _L)]
            gyv = gy_v[pl.ds(s, _L)]
            ix = (gxv + 1.0) * (_W * 0.5) - 0.5
            iy = (gyv + 1.0) * (_H * 0.5) - 0.5
            ix = jnp.minimum(jnp.maximum(ix, 0.0), _W - 1.0)
            iy = jnp.minimum(jnp.maximum(iy, 0.0), _H - 1.0)
            ix0 = ix.astype(jnp.int32)
            iy0 = iy.astype(jnp.int32)
            wx1 = ix - ix0.astype(jnp.float32)
            wy1 = iy - iy0.astype(jnp.float32)
            wx0 = 1.0 - wx1
            wy0 = 1.0 - wy1
            ix1 = jnp.minimum(ix0 + 1, _W - 1)
            iy1 = jnp.minimum(iy0 + 1, _H - 1)
            row0 = nb + iy0 * _W
            row1 = nb + iy1 * _W
            i00[pl.ds(s, _L)] = row0 + ix0
            i01[pl.ds(s, _L)] = row0 + ix1
            i10[pl.ds(s, _L)] = row1 + ix0
            i11[pl.ds(s, _L)] = row1 + ix1
            w00[pl.ds(s, _L)] = wy0 * wx0
            w01[pl.ds(s, _L)] = wy0 * wx1
            w10[pl.ds(s, _L)] = wy1 * wx0
            w11[pl.ds(s, _L)] = wy1 * wx1
        c0 = pltpu.async_copy(table.at[i00], r00, sem)
        c1 = pltpu.async_copy(table.at[i01], r01, sem)
        c2 = pltpu.async_copy(table.at[i10], r10, sem)
        c3 = pltpu.async_copy(table.at[i11], r11, sem)
        c0.wait()
        c1.wait()
        c2.wait()
        c3.wait()

        def pix(p, _):
            pv = jnp.broadcast_to(p, (_L,))
            v00 = plsc.load_gather(w00, [pv])
            v01 = plsc.load_gather(w01, [pv])
            v10 = plsc.load_gather(w10, [pv])
            v11 = plsc.load_gather(w11, [pv])
            for j in range(_C // _L):
                col = lax.iota(jnp.int32, _L) + (j * _L)
                a = plsc.load_gather(r00, [pv, col]) * v00
                a = a + plsc.load_gather(r01, [pv, col]) * v01
                a = a + plsc.load_gather(r10, [pv, col]) * v10
                a = a + plsc.load_gather(r11, [pv, col]) * v11
                plsc.store_scatter(out_v, [pv, col], a)
            return 0

        lax.fori_loop(0, _CH, pix, 0)
        pltpu.sync_copy(out_v, out_hbm.at[pl.ds(start, _CH)])
        return 0

    lax.fori_loop(0, _NCHUNK, chunk_body, 0)


def _make_warp():
    mesh = plsc.VectorSubcoreMesh(core_axis_name="c", subcore_axis_name="s")
    return pl.kernel(
        _warp_body,
        mesh=mesh,
        compiler_params=pltpu.CompilerParams(
            needs_layout_passes=False, use_tc_tiling_on_sc=False),
        out_type=jax.ShapeDtypeStruct((_B, _C), jnp.float32),
        scratch_types=[
            pltpu.VMEM((_CH,), jnp.float32),
            pltpu.VMEM((_CH,), jnp.float32),
            pltpu.VMEM((_CH,), jnp.int32),
            pltpu.VMEM((_CH,), jnp.int32),
            pltpu.VMEM((_CH,), jnp.int32),
            pltpu.VMEM((_CH,), jnp.int32),
            pltpu.VMEM((_CH,), jnp.float32),
            pltpu.VMEM((_CH,), jnp.float32),
            pltpu.VMEM((_CH,), jnp.float32),
            pltpu.VMEM((_CH,), jnp.float32),
            pltpu.VMEM((_CH, _C), jnp.float32),
            pltpu.VMEM((_CH, _C), jnp.float32),
            pltpu.VMEM((_CH, _C), jnp.float32),
            pltpu.VMEM((_CH, _C), jnp.float32),
            pltpu.VMEM((_CH, _C), jnp.float32),
            pltpu.SemaphoreType.DMA,
        ],
    )


@jax.jit
def kernel(inputs, grid):
    n, c, h, w = inputs.shape
    table = jnp.transpose(inputs, (0, 2, 3, 1)).reshape(n * h * w, c)
    g = grid.reshape(n * h * w, 2)
    gx = g[:, 0]
    gy = g[:, 1]
    out_rows = _make_warp()(table, gx, gy)
    return jnp.transpose(out_rows.reshape(n, h, w, c), (0, 3, 1, 2))


# speedup vs baseline: 1.1302x; 1.1302x over previous
import jax
import jax.numpy as jnp
from jax import lax
from jax.experimental import pallas as pl
from jax.experimental.pallas import tpu as pltpu
from jax.experimental.pallas import tpu_sc as plsc

_N, _C, _H, _W = 4, 96, 224, 224
_B = _N * _H * _W
_NW = 32
_PW = _B // _NW
_CH = 64
_NCHUNK = _PW // _CH
_L = 16

_SLOT_KEYS = ("g", "i00", "i01", "i10", "i11",
              "w00", "w01", "w10", "w11",
              "r00", "r01", "r10", "r11", "out", "sem")


def _slot_scratch():
    return [
        pltpu.VMEM((2 * _CH,), jnp.float32),
        pltpu.VMEM((_CH,), jnp.int32),
        pltpu.VMEM((_CH,), jnp.int32),
        pltpu.VMEM((_CH,), jnp.int32),
        pltpu.VMEM((_CH,), jnp.int32),
        pltpu.VMEM((_CH,), jnp.float32),
        pltpu.VMEM((_CH,), jnp.float32),
        pltpu.VMEM((_CH,), jnp.float32),
        pltpu.VMEM((_CH,), jnp.float32),
        pltpu.VMEM((_CH, _C), jnp.float32),
        pltpu.VMEM((_CH, _C), jnp.float32),
        pltpu.VMEM((_CH, _C), jnp.float32),
        pltpu.VMEM((_CH, _C), jnp.float32),
        pltpu.VMEM((_CH, _C), jnp.float32),
        pltpu.SemaphoreType.DMA,
    ]


def _warp_body(table, g_hbm, out_hbm, *scratch):
    nslot = len(_SLOT_KEYS)
    slots = [dict(zip(_SLOT_KEYS, scratch[k * nslot:(k + 1) * nslot]))
             for k in range(2)]

    wid = lax.axis_index("s") * 2 + lax.axis_index("c")
    base = wid * _PW
    nb = (wid // 8) * (_H * _W)
    iota = lax.iota(jnp.int32, _L)
    iota2 = iota * 2

    def prep_fire(ch, S):
        start = base + ch * _CH
        pltpu.sync_copy(g_hbm.at[pl.ds(2 * start, 2 * _CH)], S["g"])
        for i in range(_CH // _L):
            gxv = plsc.load_gather(S["g"], [iota2 + (2 * _L * i)])
            gyv = plsc.load_gather(S["g"], [iota2 + (2 * _L * i + 1)])
            ix = (gxv + 1.0) * (_W * 0.5) - 0.5
            iy = (gyv + 1.0) * (_H * 0.5) - 0.5
            ix = jnp.minimum(jnp.maximum(ix, 0.0), _W - 1.0)
            iy = jnp.minimum(jnp.maximum(iy, 0.0), _H - 1.0)
            ix0 = ix.astype(jnp.int32)
            iy0 = iy.astype(jnp.int32)
            wx1 = ix - ix0.astype(jnp.float32)
            wy1 = iy - iy0.astype(jnp.float32)
            wx0 = 1.0 - wx1
            wy0 = 1.0 - wy1
            ix1 = jnp.minimum(ix0 + 1, _W - 1)
            iy1 = jnp.minimum(iy0 + 1, _H - 1)
            row0 = nb + iy0 * _W
            row1 = nb + iy1 * _W
            s = i * _L
            S["i00"][pl.ds(s, _L)] = row0 + ix0
            S["i01"][pl.ds(s, _L)] = row0 + ix1
            S["i10"][pl.ds(s, _L)] = row1 + ix0
            S["i11"][pl.ds(s, _L)] = row1 + ix1
            S["w00"][pl.ds(s, _L)] = wy0 * wx0
            S["w01"][pl.ds(s, _L)] = wy0 * wx1
            S["w10"][pl.ds(s, _L)] = wy1 * wx0
            S["w11"][pl.ds(s, _L)] = wy1 * wx1
        for ib, rb in (("i00", "r00"), ("i01", "r01"),
                       ("i10", "r10"), ("i11", "r11")):
            pltpu.async_copy(table.at[S[ib]], S[rb], S["sem"])

    def wait_gathers(S):
        for ib, rb in (("i00", "r00"), ("i01", "r01"),
                       ("i10", "r10"), ("i11", "r11")):
            pltpu.make_async_copy(table.at[S[ib]], S[rb], S["sem"]).wait()

    def compute_out(ch, S):
        def pix(p, _):
            for u in range(2):
                pp = p * 2 + u
                pv = jnp.broadcast_to(pp, (_L,))
                v00 = plsc.load_gather(S["w00"], [pv])
                v01 = plsc.load_gather(S["w01"], [pv])
                v10 = plsc.load_gather(S["w10"], [pv])
                v11 = plsc.load_gather(S["w11"], [pv])
                for j in range(_C // _L):
                    cs = pl.ds(j * _L, _L)
                    a = S["r00"][pp, cs] * v00
                    a = a + S["r01"][pp, cs] * v01
                    a = a + S["r10"][pp, cs] * v10
                    a = a + S["r11"][pp, cs] * v11
                    S["out"][pp, cs] = a
            return 0

        lax.fori_loop(0, _CH // 2, pix, 0, unroll=1)
        pltpu.sync_copy(S["out"], out_hbm.at[pl.ds(base + ch * _CH, _CH)])

    prep_fire(0, slots[0])

    def step(j, _):
        prep_fire(2 * j + 1, slots[1])
        wait_gathers(slots[0])
        compute_out(2 * j, slots[0])

        @pl.when(j < _NCHUNK // 2 - 1)
        def _():
            prep_fire(2 * j + 2, slots[0])

        wait_gathers(slots[1])
        compute_out(2 * j + 1, slots[1])
        return 0

    lax.fori_loop(0, _NCHUNK // 2, step, 0)


def _make_warp():
    mesh = plsc.VectorSubcoreMesh(core_axis_name="c", subcore_axis_name="s")
    return pl.kernel(
        _warp_body,
        mesh=mesh,
        compiler_params=pltpu.CompilerParams(
            needs_layout_passes=False, use_tc_tiling_on_sc=False),
        out_type=jax.ShapeDtypeStruct((_B, _C), jnp.float32),
        scratch_types=_slot_scratch() + _slot_scratch(),
    )


@jax.jit
def kernel(inputs, grid):
    n, c, h, w = inputs.shape
    table = jnp.transpose(inputs, (0, 2, 3, 1)).reshape(n * h * w, c)
    g = grid.reshape(n * h * w * 2)
    out_rows = _make_warp()(table, g)
    return jnp.transpose(out_rows.reshape(n, h, w, c), (0, 3, 1, 2))
